# baseline (speedup 1.0000x reference)
import functools

import jax
import jax.numpy as jnp
import numpy as np
from jax import lax
from jax.experimental import pallas as pl
from jax.experimental.pallas import tpu as pltpu
from jax.experimental.pallas import tpu_sc as plsc

_F = 26
_D = 16
_ND = 2 * _D
_B = 4096
_VOCAB = 100000
_TOTAL = _F * _VOCAB
_NC, _NS = 2, 16
_NW = _NC * _NS
_RPW = _B // _NW
_CH = 16
_NCH = _RPW // _CH
_EPD = _F * _RPW

_OFFSETS = (np.arange(_F, dtype=np.int32) * _VOCAB)[None, :]

_mesh = plsc.VectorSubcoreMesh(core_axis_name="c", subcore_axis_name="s")


def _body(embt_hbm, fcw_hbm, bias_hbm, idxf_hbm, out_hbm,
          idxf_v, fc_v, g_v, bias_v, y_v, sem_f, sem0, sem1):
    wid = lax.axis_index("s") * _NC + lax.axis_index("c")

    pltpu.sync_copy(idxf_hbm.at[wid], idxf_v)
    pltpu.sync_copy(bias_hbm, bias_v)

    for f in range(_F):
        pltpu.make_async_copy(
            fcw_hbm.at[idxf_v.at[f]], fc_v.at[pl.ds(f * _RPW, _RPW)], sem_f
        ).start()

    sems = (sem0, sem1)

    def issue(d, slot):
        for f in range(_F):
            pltpu.make_async_copy(
                embt_hbm.at[d].at[idxf_v.at[f]],
                g_v.at[slot, pl.ds(f * _RPW, _RPW)],
                sems[slot],
            ).start()

    def wait_g(slot):
        for f in range(_F):
            pltpu.make_async_copy(
                embt_hbm.at[0].at[pl.ds(0, _RPW)],
                g_v.at[slot, pl.ds(f * _RPW, _RPW)],
                sems[slot],
            ).wait()

    issue(0, 0)
    issue(1, 1)

    for f in range(_F):
        pltpu.make_async_copy(
            fcw_hbm.at[pl.ds(0, _RPW)], fc_v.at[pl.ds(f * _RPW, _RPW)], sem_f
        ).wait()

    zero = jnp.zeros((_CH,), jnp.float32)

    def accum(slot, c):
        a1 = zero
        a2 = zero
        a3 = zero
        for f in range(_F):
            v = g_v[slot, pl.ds(f * _RPW + c * _CH, _CH)]
            a1 = a1 + v
            sq = v * v
            a2 = a2 + sq
            a3 = a3 + sq * v
        return a1, a2, a3

    def lo_pair(k, tot):
        d0 = 2 * k
        wait_g(0)
        new = []
        for c in range(_NCH):
            a1, a2, _ = accum(0, c)
            new.append(tot[c] + 0.5 * (a1 * a1 - a2))
        tot = tuple(new)
        issue(d0 + 2, 0)
        wait_g(1)
        new = []
        for c in range(_NCH):
            a1, a2, _ = accum(1, c)
            new.append(tot[c] + 0.5 * (a1 * a1 - a2))
        tot = tuple(new)
        issue(d0 + 3, 1)
        return tot

    def hi_pair(k, tot):
        d0 = 2 * k
        wait_g(0)
        new = []
        for c in range(_NCH):
            a1, a2, a3 = accum(0, c)
            new.append(tot[c] + (1.0 / 6.0) * (a1 * (a1 * a1 - 3.0 * a2) + 2.0 * a3))
        tot = tuple(new)

        @pl.when(d0 + 2 < _ND)
        def _():
            issue(d0 + 2, 0)

        wait_g(1)
        new = []
        for c in range(_NCH):
            a1, a2, a3 = accum(1, c)
            new.append(tot[c] + (1.0 / 6.0) * (a1 * (a1 * a1 - 3.0 * a2) + 2.0 * a3))
        tot = tuple(new)

        @pl.when(d0 + 3 < _ND)
        def _():
            issue(d0 + 3, 1)

        return tot

    tot = (zero,) * _NCH
    tot = lax.fori_loop(0, _D // 2, lo_pair, tot)
    tot = lax.fori_loop(_D // 2, _ND // 2, hi_pair, tot)

    bias = bias_v[...]
    for c in range(_NCH):
        lin = bias
        for f in range(_F):
            lin = lin + fc_v[pl.ds(f * _RPW + c * _CH, _CH)]
        y = lin + tot[c]
        y_v[pl.ds(c * _CH, _CH)] = 1.0 / (1.0 + jnp.exp(-y))

    pltpu.sync_copy(y_v, out_hbm.at[pl.ds(wid * _RPW, _RPW)])


_fm_kernel = functools.partial(
    pl.kernel,
    out_type=jax.ShapeDtypeStruct((_B,), jnp.float32),
    mesh=_mesh,
    scratch_types=[
        pltpu.VMEM((_F, _RPW), jnp.int32),
        pltpu.VMEM((_EPD,), jnp.float32),
        pltpu.VMEM((2, _EPD), jnp.float32),
        pltpu.VMEM((_CH,), jnp.float32),
        pltpu.VMEM((_RPW,), jnp.float32),
        pltpu.SemaphoreType.DMA,
        pltpu.SemaphoreType.DMA,
        pltpu.SemaphoreType.DMA,
    ],
    compiler_params=pltpu.CompilerParams(use_tc_tiling_on_sc=False),
)(_body)


@jax.jit
def kernel(x, fc_weight, fc_bias, emb_weight):
    xo = x.astype(jnp.int32) + jnp.asarray(_OFFSETS)
    xof = jnp.transpose(xo.reshape(_NW, _RPW, _F), (0, 2, 1))
    fcw = fc_weight.reshape(-1)
    bias16 = jnp.broadcast_to(fc_bias.astype(jnp.float32), (_CH,))
    embt = emb_weight.T
    return _fm_kernel(embt, fcw, bias16, xof)


# speedup vs baseline: 9.1431x; 9.1431x over previous
import functools

import jax
import jax.numpy as jnp
import numpy as np
from jax import lax
from jax.experimental import pallas as pl
from jax.experimental.pallas import tpu as pltpu
from jax.experimental.pallas import tpu_sc as plsc

_F = 26
_D = 16
_ND = 2 * _D
_B = 4096
_VOCAB = 100000
_TOTAL = _F * _VOCAB
_PADT = 2600960
_FLAT = _ND * _PADT
_NC, _NS = 2, 16
_NW = _NC * _NS
_RPW = _B // _NW
_CH = 16
_NCH = _RPW // _CH
_EPD = _F * _RPW

_BS = 130048
_NWIN = _PADT // _BS

_OFFSETS = (np.arange(_F, dtype=np.int32) * _VOCAB)[None, :]
_DIM_OFF = (np.arange(_ND, dtype=np.int32) * _PADT)[None, :, None, None]

_mesh = plsc.VectorSubcoreMesh(core_axis_name="c", subcore_axis_name="s")


def _copy_body(i_ref, o_ref):
    o_ref[...] = i_ref[pl.program_id(2), :]


_relayout = pl.pallas_call(
    _copy_body,
    grid=(_ND // 8, _NWIN, 8),
    in_specs=[pl.BlockSpec((8, _BS), lambda g, w, k: (g, w))],
    out_specs=pl.BlockSpec((_BS,), lambda g, w, k: ((g * 8 + k) * _NWIN + w,)),
    out_shape=jax.ShapeDtypeStruct((_FLAT,), jnp.float32),
)


def _body(embf_hbm, fcw_hbm, bias_hbm, idxa_hbm, idxf_hbm, out_hbm,
          idxf_v, fc_v, idxd_v, g_v, bias_v, y_v, sem_f, sem0, sem1):
    wid = lax.axis_index("s") * _NC + lax.axis_index("c")

    pltpu.sync_copy(idxf_hbm.at[wid], idxf_v)
    pltpu.sync_copy(bias_hbm, bias_v)

    for f in range(_F):
        pltpu.make_async_copy(
            fcw_hbm.at[idxf_v.at[f]], fc_v.at[pl.ds(f * _RPW, _RPW)], sem_f
        ).start()

    sems = (sem0, sem1)

    def stage_and_issue(d, slot):
        pltpu.sync_copy(idxa_hbm.at[wid, d], idxd_v.at[slot])
        for f in range(_F):
            pltpu.make_async_copy(
                embf_hbm.at[idxd_v.at[slot, f]],
                g_v.at[slot, pl.ds(f * _RPW, _RPW)],
                sems[slot],
            ).start()

    def wait_g(slot):
        for f in range(_F):
            pltpu.make_async_copy(
                embf_hbm.at[pl.ds(0, _RPW)],
                g_v.at[slot, pl.ds(f * _RPW, _RPW)],
                sems[slot],
            ).wait()

    stage_and_issue(0, 0)
    stage_and_issue(1, 1)

    for f in range(_F):
        pltpu.make_async_copy(
            fcw_hbm.at[pl.ds(0, _RPW)], fc_v.at[pl.ds(f * _RPW, _RPW)], sem_f
        ).wait()

    zero = jnp.zeros((_CH,), jnp.float32)

    def accum(slot, c):
        a1 = zero
        a2 = zero
        a3 = zero
        for f in range(_F):
            v = g_v[slot, pl.ds(f * _RPW + c * _CH, _CH)]
            a1 = a1 + v
            sq = v * v
            a2 = a2 + sq
            a3 = a3 + sq * v
        return a1, a2, a3

    def lo_pair(k, tot):
        d0 = 2 * k
        wait_g(0)
        new = []
        for c in range(_NCH):
            a1, a2, _ = accum(0, c)
            new.append(tot[c] + 0.5 * (a1 * a1 - a2))
        tot = tuple(new)
        stage_and_issue(d0 + 2, 0)
        wait_g(1)
        new = []
        for c in range(_NCH):
            a1, a2, _ = accum(1, c)
            new.append(tot[c] + 0.5 * (a1 * a1 - a2))
        tot = tuple(new)
        stage_and_issue(d0 + 3, 1)
        return tot

    def hi_pair(k, tot):
        d0 = 2 * k
        wait_g(0)
        new = []
        for c in range(_NCH):
            a1, a2, a3 = accum(0, c)
            new.append(tot[c] + (1.0 / 6.0) * (a1 * (a1 * a1 - 3.0 * a2) + 2.0 * a3))
        tot = tuple(new)

        @pl.when(d0 + 2 < _ND)
        def _():
            stage_and_issue(d0 + 2, 0)

        wait_g(1)
        new = []
        for c in range(_NCH):
            a1, a2, a3 = accum(1, c)
            new.append(tot[c] + (1.0 / 6.0) * (a1 * (a1 * a1 - 3.0 * a2) + 2.0 * a3))
        tot = tuple(new)

        @pl.when(d0 + 3 < _ND)
        def _():
            stage_and_issue(d0 + 3, 1)

        return tot

    tot = (zero,) * _NCH
    tot = lax.fori_loop(0, _D // 2, lo_pair, tot)
    tot = lax.fori_loop(_D // 2, _ND // 2, hi_pair, tot)

    bias = bias_v[...]
    for c in range(_NCH):
        lin = bias
        for f in range(_F):
            lin = lin + fc_v[pl.ds(f * _RPW + c * _CH, _CH)]
        y = lin + tot[c]
        y_v[pl.ds(c * _CH, _CH)] = 1.0 / (1.0 + jnp.exp(-y))

    pltpu.sync_copy(y_v, out_hbm.at[pl.ds(wid * _RPW, _RPW)])


_fm_kernel = functools.partial(
    pl.kernel,
    out_type=jax.ShapeDtypeStruct((_B,), jnp.float32),
    mesh=_mesh,
    scratch_types=[
        pltpu.VMEM((_F, _RPW), jnp.int32),
        pltpu.VMEM((_EPD,), jnp.float32),
        pltpu.VMEM((2, _F, _RPW), jnp.int32),
        pltpu.VMEM((2, _EPD), jnp.float32),
        pltpu.VMEM((_CH,), jnp.float32),
        pltpu.VMEM((_RPW,), jnp.float32),
        pltpu.SemaphoreType.DMA,
        pltpu.SemaphoreType.DMA,
        pltpu.SemaphoreType.DMA,
    ],
    compiler_params=pltpu.CompilerParams(use_tc_tiling_on_sc=False),
)(_body)


@jax.jit
def kernel(x, fc_weight, fc_bias, emb_weight):
    xo = x.astype(jnp.int32) + jnp.asarray(_OFFSETS)
    xof = jnp.transpose(xo.reshape(_NW, _RPW, _F), (0, 2, 1))
    idxa = xof[:, None, :, :] + jnp.asarray(_DIM_OFF)
    fcw = fc_weight.reshape(-1)
    bias16 = jnp.broadcast_to(fc_bias.astype(jnp.float32), (_CH,))
    embf = _relayout(emb_weight.T)
    return _fm_kernel(embf, fcw, bias16, idxa, xof)
